# baseline (speedup 1.0000x reference)
import functools

import jax
import jax.numpy as jnp
from jax import lax
from jax.experimental import pallas as pl
from jax.experimental.pallas import tpu as pltpu
from jax.experimental.pallas import tpu_sc as plsc

_K = 8192
_D = 32
_BETA = 0.25
_ALPHA = 1.0
_BN = 512
_KC = 2048


def _argmin_tile(zf_ref, zsq_ref, wt_ref, wsq_ref, tok_ref):
    zf = zf_ref[...].astype(jnp.float32)
    acc_v = jnp.full((_BN,), jnp.inf, jnp.float32)
    acc_i = jnp.zeros((_BN,), jnp.int32)
    for c in range(_K // _KC):
        wt = wt_ref[:, c * _KC:(c + 1) * _KC].astype(jnp.float32)
        mm = jnp.dot(zf, wt, preferred_element_type=jnp.float32)
        t = zsq_ref[...] + wsq_ref[:, c * _KC:(c + 1) * _KC]
        v = t - 2.0 * mm
        cv = jnp.min(v, axis=1)
        lane = lax.broadcasted_iota(jnp.int32, (_BN, _KC), 1)
        ci = jnp.min(jnp.where(v == cv[:, None], lane, _K), axis=1) + c * _KC
        keep = acc_v <= cv
        acc_i = jnp.where(keep, acc_i, ci)
        acc_v = jnp.where(keep, acc_v, cv).astype(jnp.bfloat16).astype(jnp.float32)
    tok_ref[...] = acc_i.reshape(tok_ref.shape)


def _make_sc_tail(n):
    info = plsc.get_sparse_core_info()
    nc, ns = info.num_cores, info.num_subcores
    nw = nc * ns
    bpw = n // nw
    bins = _K // ns

    mesh = plsc.VectorSubcoreMesh(core_axis_name="c", subcore_axis_name="s")

    @functools.partial(
        pl.kernel, mesh=mesh,
        out_type=[jax.ShapeDtypeStruct((n, 128), jnp.float32),
                  jax.ShapeDtypeStruct((nc * _K,), jnp.float32)],
        scratch_types=[
            pltpu.VMEM((bpw,), jnp.int32),
            pltpu.VMEM((bpw, 128), jnp.float32),
            pltpu.VMEM((bpw,), jnp.float32),
            pltpu.VMEM((bins,), jnp.float32),
            pltpu.VMEM_SHARED((_K,), jnp.float32),
            pltpu.SemaphoreType.DMA,
        ],
    )
    def sc_tail(tok_hbm, w_hbm, zq_hbm, hist_hbm,
                idx_v, rows_v, ones_v, zeros_v, hist_sh, sem):
        cid = lax.axis_index("c")
        sid = lax.axis_index("s")
        wid = sid * nc + cid
        base = wid * bpw
        pltpu.sync_copy(tok_hbm.at[pl.ds(base, bpw)], idx_v)
        pltpu.async_copy(w_hbm.at[idx_v], rows_v, sem).wait()
        pltpu.sync_copy(rows_v, zq_hbm.at[pl.ds(base, bpw)])
        for i in range(bpw // 16):
            ones_v[pl.ds(i * 16, 16)] = jnp.full((16,), 1.0, jnp.float32)
        for i in range(bins // 16):
            zeros_v[pl.ds(i * 16, 16)] = jnp.zeros((16,), jnp.float32)
        pltpu.sync_copy(zeros_v, hist_sh.at[pl.ds(sid * bins, bins)])
        plsc.subcore_barrier()
        pltpu.sync_copy(ones_v, hist_sh.at[idx_v], add=True)
        plsc.subcore_barrier()
        pltpu.sync_copy(hist_sh.at[pl.ds(sid * bins, bins)],
                        hist_hbm.at[pl.ds(cid * _K + sid * bins, bins)])

    return sc_tail


def _finalize(zp_ref, zq_ref, hist2_ref, out_ref, loss_ref, qe_ref, util_ref,
              perp_ref):
    zp = zp_ref[...]
    zq = zq_ref[:, : _D]
    diff = zq - zp
    sq = diff * diff
    n, dd = zp.shape
    m = jnp.sum(sq) / (n * dd)
    loss_ref[...] = (_BETA * m + _ALPHA * m).reshape(1, 1)
    qe_ref[...] = (jnp.sum(jnp.sum(sq, axis=1)) / n).reshape(1, 1)
    hist = hist2_ref[0, :] + hist2_ref[1, :]
    util_ref[...] = (jnp.sum((hist > 0).astype(jnp.float32)) / _K).reshape(1, 1)
    p = hist / jnp.sum(hist)
    perp_ref[...] = jnp.exp(-jnp.sum(p * jnp.log(p + 1e-10))).reshape(1, 1)
    out_ref[...] = zp + (zq - zp)


def kernel(z, emb_weight):
    b, ch, h, wd = z.shape
    zp = jnp.transpose(z, (0, 2, 3, 1))
    zf = zp.reshape(-1, _D)
    n = zf.shape[0]
    w = jax.lax.stop_gradient(emb_weight)
    zsq = jnp.sum(zf ** 2, axis=1, keepdims=True)
    wsq = jnp.sum(w ** 2, axis=1).reshape(1, _K)
    zf_bf = zf.astype(jnp.bfloat16)
    wt_bf = w.astype(jnp.bfloat16).T
    nb = n // _BN
    token2 = pl.pallas_call(
        _argmin_tile,
        grid=(nb,),
        in_specs=[
            pl.BlockSpec((_BN, _D), lambda i: (i, 0)),
            pl.BlockSpec((_BN, 1), lambda i: (i, 0)),
            pl.BlockSpec((_D, _K), lambda i: (0, 0)),
            pl.BlockSpec((1, _K), lambda i: (0, 0)),
        ],
        out_specs=pl.BlockSpec((_BN, 1), lambda i: (i, 0)),
        out_shape=jax.ShapeDtypeStruct((n, 1), jnp.int32),
        compiler_params=pltpu.CompilerParams(
            dimension_semantics=("parallel",)),
    )(zf_bf, zsq, wt_bf, wsq)
    token = token2.reshape(-1)

    w_pad = jnp.pad(emb_weight, ((0, 0), (0, 128 - _D)))
    zq_pad, hist_flat = _make_sc_tail(n)(token, w_pad)
    hist2 = hist_flat.reshape(2, _K)

    out_flat, loss, qe, util, perp = pl.pallas_call(
        _finalize,
        in_specs=[
            pl.BlockSpec((n, _D), lambda: (0, 0)),
            pl.BlockSpec((n, 128), lambda: (0, 0)),
            pl.BlockSpec((2, _K), lambda: (0, 0)),
        ],
        out_shape=[
            jax.ShapeDtypeStruct((n, _D), jnp.float32),
            jax.ShapeDtypeStruct((1, 1), jnp.float32),
            jax.ShapeDtypeStruct((1, 1), jnp.float32),
            jax.ShapeDtypeStruct((1, 1), jnp.float32),
            jax.ShapeDtypeStruct((1, 1), jnp.float32),
        ],
    )(zf, zq_pad, hist2)

    out = out_flat.reshape(b, h, wd, ch).transpose(0, 3, 1, 2)
    return (out, loss.reshape(()), qe.reshape(()), util.reshape(()),
            perp.reshape(()))


# speedup vs baseline: 1.0201x; 1.0201x over previous
import functools

import jax
import jax.numpy as jnp
from jax import lax
from jax.experimental import pallas as pl
from jax.experimental.pallas import tpu as pltpu
from jax.experimental.pallas import tpu_sc as plsc

_K = 8192
_D = 32
_BETA = 0.25
_ALPHA = 1.0
_BN = 512
_KC = 2048


def _argmin_tile(zf_ref, zsq_ref, wt_ref, wsq_ref, tok_ref):
    zf = zf_ref[...].astype(jnp.float32)
    acc_v = jnp.full((_BN,), jnp.inf, jnp.float32)
    acc_i = jnp.zeros((_BN,), jnp.int32)
    lane = lax.broadcasted_iota(jnp.int32, (_BN, _KC), 1)
    for c in range(_K // _KC):
        wt = wt_ref[:, c * _KC:(c + 1) * _KC].astype(jnp.float32)
        mm = jnp.dot(zf, wt, preferred_element_type=jnp.float32)
        t = zsq_ref[...] + wsq_ref[:, c * _KC:(c + 1) * _KC]
        v = t - mm
        cv = jnp.min(v, axis=1)
        ci = jnp.min(jnp.where(v == cv[:, None], lane, _K), axis=1) + c * _KC
        keep = acc_v <= cv
        acc_i = jnp.where(keep, acc_i, ci)
        acc_v = jnp.where(keep, acc_v, cv).astype(jnp.bfloat16).astype(jnp.float32)
    tok_ref[...] = acc_i.reshape(tok_ref.shape)


def _make_sc_tail(n):
    info = plsc.get_sparse_core_info()
    nc, ns = info.num_cores, info.num_subcores
    nw = nc * ns
    bpw = n // nw
    bins = _K // ns

    mesh = plsc.VectorSubcoreMesh(core_axis_name="c", subcore_axis_name="s")

    @functools.partial(
        pl.kernel, mesh=mesh,
        out_type=[jax.ShapeDtypeStruct((n, 128), jnp.float32),
                  jax.ShapeDtypeStruct((nc * _K,), jnp.float32)],
        scratch_types=[
            pltpu.VMEM((bpw,), jnp.int32),
            pltpu.VMEM((bpw, 128), jnp.float32),
            pltpu.VMEM((bpw,), jnp.float32),
            pltpu.VMEM((bins,), jnp.float32),
            pltpu.VMEM_SHARED((_K,), jnp.float32),
            pltpu.SemaphoreType.DMA,
        ],
    )
    def sc_tail(tok_hbm, w_hbm, zq_hbm, hist_hbm,
                idx_v, rows_v, ones_v, zeros_v, hist_sh, sem):
        cid = lax.axis_index("c")
        sid = lax.axis_index("s")
        wid = sid * nc + cid
        base = wid * bpw
        pltpu.sync_copy(tok_hbm.at[pl.ds(base, bpw)], idx_v)
        pltpu.async_copy(w_hbm.at[idx_v], rows_v, sem).wait()
        pltpu.sync_copy(rows_v, zq_hbm.at[pl.ds(base, bpw)])
        for i in range(bpw // 16):
            ones_v[pl.ds(i * 16, 16)] = jnp.full((16,), 1.0, jnp.float32)
        for i in range(bins // 16):
            zeros_v[pl.ds(i * 16, 16)] = jnp.zeros((16,), jnp.float32)
        pltpu.sync_copy(zeros_v, hist_sh.at[pl.ds(sid * bins, bins)])
        plsc.subcore_barrier()
        pltpu.sync_copy(ones_v, hist_sh.at[idx_v], add=True)
        plsc.subcore_barrier()
        pltpu.sync_copy(hist_sh.at[pl.ds(sid * bins, bins)],
                        hist_hbm.at[pl.ds(cid * _K + sid * bins, bins)])

    return sc_tail


def _finalize(zp_ref, zq_ref, hist2_ref, out_ref, loss_ref, qe_ref, util_ref,
              perp_ref):
    zp = zp_ref[...]
    zq = zq_ref[:, : _D]
    diff = zq - zp
    sq = diff * diff
    n, dd = zp.shape
    m = jnp.sum(sq) / (n * dd)
    loss_ref[...] = (_BETA * m + _ALPHA * m).reshape(1, 1)
    qe_ref[...] = (jnp.sum(jnp.sum(sq, axis=1)) / n).reshape(1, 1)
    hist = hist2_ref[0, :] + hist2_ref[1, :]
    util_ref[...] = (jnp.sum((hist > 0).astype(jnp.float32)) / _K).reshape(1, 1)
    p = hist / jnp.sum(hist)
    perp_ref[...] = jnp.exp(-jnp.sum(p * jnp.log(p + 1e-10))).reshape(1, 1)
    out_ref[...] = zp + (zq - zp)


def kernel(z, emb_weight):
    b, ch, h, wd = z.shape
    zp = jnp.transpose(z, (0, 2, 3, 1))
    zf = zp.reshape(-1, _D)
    n = zf.shape[0]
    w = jax.lax.stop_gradient(emb_weight)
    zsq = jnp.sum(zf ** 2, axis=1, keepdims=True)
    wsq = jnp.sum(w ** 2, axis=1).reshape(1, _K)
    zf_bf = (2.0 * zf).astype(jnp.bfloat16)
    wt_bf = w.astype(jnp.bfloat16).T
    nb = n // _BN
    token2 = pl.pallas_call(
        _argmin_tile,
        grid=(nb,),
        in_specs=[
            pl.BlockSpec((_BN, _D), lambda i: (i, 0)),
            pl.BlockSpec((_BN, 1), lambda i: (i, 0)),
            pl.BlockSpec((_D, _K), lambda i: (0, 0)),
            pl.BlockSpec((1, _K), lambda i: (0, 0)),
        ],
        out_specs=pl.BlockSpec((_BN, 1), lambda i: (i, 0)),
        out_shape=jax.ShapeDtypeStruct((n, 1), jnp.int32),
        compiler_params=pltpu.CompilerParams(
            dimension_semantics=("parallel",)),
    )(zf_bf, zsq, wt_bf, wsq)
    token = token2.reshape(-1)

    w_pad = jnp.pad(emb_weight, ((0, 0), (0, 128 - _D)))
    zq_pad, hist_flat = _make_sc_tail(n)(token, w_pad)
    hist2 = hist_flat.reshape(2, _K)

    out_flat, loss, qe, util, perp = pl.pallas_call(
        _finalize,
        in_specs=[
            pl.BlockSpec((n, _D), lambda: (0, 0)),
            pl.BlockSpec((n, 128), lambda: (0, 0)),
            pl.BlockSpec((2, _K), lambda: (0, 0)),
        ],
        out_shape=[
            jax.ShapeDtypeStruct((n, _D), jnp.float32),
            jax.ShapeDtypeStruct((1, 1), jnp.float32),
            jax.ShapeDtypeStruct((1, 1), jnp.float32),
            jax.ShapeDtypeStruct((1, 1), jnp.float32),
            jax.ShapeDtypeStruct((1, 1), jnp.float32),
        ],
    )(zf, zq_pad, hist2)

    out = out_flat.reshape(b, h, wd, ch).transpose(0, 3, 1, 2)
    return (out, loss.reshape(()), qe.reshape(()), util.reshape(()),
            perp.reshape(()))


# speedup vs baseline: 1.0434x; 1.0229x over previous
import functools

import jax
import jax.numpy as jnp
from jax import lax
from jax.experimental import pallas as pl
from jax.experimental.pallas import tpu as pltpu
from jax.experimental.pallas import tpu_sc as plsc

_K = 8192
_D = 32
_BETA = 0.25
_ALPHA = 1.0
_BN = 1024
_KC = 2048


def _argmin_tile(zf_ref, zsq_ref, wt_ref, wsq_ref, tok_ref):
    zf = zf_ref[...].astype(jnp.float32)
    acc_v = jnp.full((_BN,), jnp.inf, jnp.float32)
    acc_i = jnp.zeros((_BN,), jnp.int32)
    lane = lax.broadcasted_iota(jnp.int32, (_BN, _KC), 1)
    for c in range(_K // _KC):
        wt = wt_ref[:, c * _KC:(c + 1) * _KC].astype(jnp.float32)
        mm = jnp.dot(zf, wt, preferred_element_type=jnp.float32)
        t = zsq_ref[...] + wsq_ref[:, c * _KC:(c + 1) * _KC]
        v = t - mm
        cv = jnp.min(v, axis=1)
        ci = jnp.min(jnp.where(v == cv[:, None], lane, _K), axis=1) + c * _KC
        keep = acc_v <= cv
        acc_i = jnp.where(keep, acc_i, ci)
        acc_v = jnp.where(keep, acc_v, cv).astype(jnp.bfloat16).astype(jnp.float32)
    tok_ref[...] = acc_i.reshape(tok_ref.shape)


def _make_sc_tail(n):
    info = plsc.get_sparse_core_info()
    nc, ns = info.num_cores, info.num_subcores
    nw = nc * ns
    bpw = n // nw
    bins = _K // ns

    mesh = plsc.VectorSubcoreMesh(core_axis_name="c", subcore_axis_name="s")

    @functools.partial(
        pl.kernel, mesh=mesh,
        out_type=[jax.ShapeDtypeStruct((n, 128), jnp.float32),
                  jax.ShapeDtypeStruct((nc * _K,), jnp.float32)],
        scratch_types=[
            pltpu.VMEM((bpw,), jnp.int32),
            pltpu.VMEM((bpw, 128), jnp.float32),
            pltpu.VMEM((bpw,), jnp.float32),
            pltpu.VMEM((bins,), jnp.float32),
            pltpu.VMEM_SHARED((_K,), jnp.float32),
            pltpu.SemaphoreType.DMA,
        ],
    )
    def sc_tail(tok_hbm, w_hbm, zq_hbm, hist_hbm,
                idx_v, rows_v, ones_v, zeros_v, hist_sh, sem):
        cid = lax.axis_index("c")
        sid = lax.axis_index("s")
        wid = sid * nc + cid
        base = wid * bpw
        pltpu.sync_copy(tok_hbm.at[pl.ds(base, bpw)], idx_v)
        pltpu.async_copy(w_hbm.at[idx_v], rows_v, sem).wait()
        pltpu.sync_copy(rows_v, zq_hbm.at[pl.ds(base, bpw)])
        for i in range(bpw // 16):
            ones_v[pl.ds(i * 16, 16)] = jnp.full((16,), 1.0, jnp.float32)
        for i in range(bins // 16):
            zeros_v[pl.ds(i * 16, 16)] = jnp.zeros((16,), jnp.float32)
        pltpu.sync_copy(zeros_v, hist_sh.at[pl.ds(sid * bins, bins)])
        plsc.subcore_barrier()
        pltpu.sync_copy(ones_v, hist_sh.at[idx_v], add=True)
        plsc.subcore_barrier()
        pltpu.sync_copy(hist_sh.at[pl.ds(sid * bins, bins)],
                        hist_hbm.at[pl.ds(cid * _K + sid * bins, bins)])

    return sc_tail


def _finalize(zp_ref, zq_ref, hist2_ref, out_ref, loss_ref, qe_ref, util_ref,
              perp_ref):
    zp = zp_ref[...]
    zq = zq_ref[:, : _D]
    diff = zq - zp
    sq = diff * diff
    n, dd = zp.shape
    m = jnp.sum(sq) / (n * dd)
    loss_ref[...] = (_BETA * m + _ALPHA * m).reshape(1, 1)
    qe_ref[...] = (jnp.sum(jnp.sum(sq, axis=1)) / n).reshape(1, 1)
    hist = hist2_ref[0, :] + hist2_ref[1, :]
    util_ref[...] = (jnp.sum((hist > 0).astype(jnp.float32)) / _K).reshape(1, 1)
    p = hist / jnp.sum(hist)
    perp_ref[...] = jnp.exp(-jnp.sum(p * jnp.log(p + 1e-10))).reshape(1, 1)
    out_ref[...] = zp + (zq - zp)


def kernel(z, emb_weight):
    b, ch, h, wd = z.shape
    zp = jnp.transpose(z, (0, 2, 3, 1))
    zf = zp.reshape(-1, _D)
    n = zf.shape[0]
    w = jax.lax.stop_gradient(emb_weight)
    zsq = jnp.sum(zf ** 2, axis=1, keepdims=True)
    wsq = jnp.sum(w ** 2, axis=1).reshape(1, _K)
    zf_bf = (2.0 * zf).astype(jnp.bfloat16)
    wt_bf = w.astype(jnp.bfloat16).T
    nb = n // _BN
    token2 = pl.pallas_call(
        _argmin_tile,
        grid=(nb,),
        in_specs=[
            pl.BlockSpec((_BN, _D), lambda i: (i, 0)),
            pl.BlockSpec((_BN, 1), lambda i: (i, 0)),
            pl.BlockSpec((_D, _K), lambda i: (0, 0)),
            pl.BlockSpec((1, _K), lambda i: (0, 0)),
        ],
        out_specs=pl.BlockSpec((_BN, 1), lambda i: (i, 0)),
        out_shape=jax.ShapeDtypeStruct((n, 1), jnp.int32),
        compiler_params=pltpu.CompilerParams(
            dimension_semantics=("parallel",)),
    )(zf_bf, zsq, wt_bf, wsq)
    token = token2.reshape(-1)

    w_pad = jnp.pad(emb_weight, ((0, 0), (0, 128 - _D)))
    zq_pad, hist_flat = _make_sc_tail(n)(token, w_pad)
    hist2 = hist_flat.reshape(2, _K)

    out_flat, loss, qe, util, perp = pl.pallas_call(
        _finalize,
        in_specs=[
            pl.BlockSpec((n, _D), lambda: (0, 0)),
            pl.BlockSpec((n, 128), lambda: (0, 0)),
            pl.BlockSpec((2, _K), lambda: (0, 0)),
        ],
        out_shape=[
            jax.ShapeDtypeStruct((n, _D), jnp.float32),
            jax.ShapeDtypeStruct((1, 1), jnp.float32),
            jax.ShapeDtypeStruct((1, 1), jnp.float32),
            jax.ShapeDtypeStruct((1, 1), jnp.float32),
            jax.ShapeDtypeStruct((1, 1), jnp.float32),
        ],
    )(zf, zq_pad, hist2)

    out = out_flat.reshape(b, h, wd, ch).transpose(0, 3, 1, 2)
    return (out, loss.reshape(()), qe.reshape(()), util.reshape(()),
            perp.reshape(()))
